# initial kernel scaffold (unmeasured)
import jax
import jax.numpy as jnp
from jax import lax
from jax.experimental import pallas as pl
from jax.experimental.pallas import tpu as pltpu

N_DEV = 8
M_CHUNK = 1024


def kernel(x, W1, W2):
    M, K = x.shape
    _, D = W1.shape
    _, F = W2.shape
    n_steps = N_DEV - 1

    def body(x_hbm, w1_ref, w2_ref, out_ref,
             xblk_ref, part_ref, sbuf_ref, comm_ref,
             load_sem, send_sems, recv_sems, credit_sem):
        my = lax.axis_index("i")
        left = lax.rem(my + N_DEV - 1, N_DEV)
        right = lax.rem(my + 1, N_DEV)

        barrier = pltpu.get_barrier_semaphore()
        for nbr in (left, right):
            pl.semaphore_signal(barrier, inc=1, device_id=(nbr,),
                                device_id_type=pl.DeviceIdType.MESH)
        pl.semaphore_wait(barrier, 2)

        def compute_partial(c, dst_ref):
            cp = pltpu.make_async_copy(
                x_hbm.at[pl.ds(c * M_CHUNK, M_CHUNK), :], xblk_ref, load_sem)
            cp.start()
            cp.wait()
            dst_ref[...] = jnp.dot(xblk_ref[...], w1_ref[...],
                                   preferred_element_type=jnp.float32)

        def signal_credit():
            pl.semaphore_signal(credit_sem, inc=1, device_id=(left,),
                                device_id_type=pl.DeviceIdType.MESH)

        compute_partial(my, sbuf_ref.at[0])
        for s in range(n_steps):
            slot = s % 2
            if s >= 2:
                pl.semaphore_wait(credit_sem, 1)
            rdma = pltpu.make_async_remote_copy(
                src_ref=sbuf_ref.at[slot],
                dst_ref=comm_ref.at[slot],
                send_sem=send_sems.at[slot],
                recv_sem=recv_sems.at[slot],
                device_id=(right,),
                device_id_type=pl.DeviceIdType.MESH,
            )
            rdma.start()
            c = lax.rem(my + N_DEV - s - 1, N_DEV)
            compute_partial(c, part_ref)
            rdma.wait()
            sbuf_ref[1 - slot] = comm_ref[slot] + part_ref[...]
            signal_credit()

        for t in range(n_steps):
            s = n_steps + t
            slot = s % 2
            pl.semaphore_wait(credit_sem, 1)
            src_ref = sbuf_ref.at[1] if t == 0 else comm_ref.at[(s - 1) % 2]
            rdma = pltpu.make_async_remote_copy(
                src_ref=src_ref,
                dst_ref=comm_ref.at[slot],
                send_sem=send_sems.at[slot],
                recv_sem=recv_sems.at[slot],
                device_id=(right,),
                device_id_type=pl.DeviceIdType.MESH,
            )
            rdma.start()
            c_held = lax.rem(my + 1 - t + N_DEV, N_DEV)
            out_ref[pl.ds(c_held * M_CHUNK, M_CHUNK), :] = jnp.dot(
                src_ref[...], w2_ref[...], preferred_element_type=jnp.float32)
            rdma.wait()
            if 1 <= t <= 5:
                signal_credit()

        c_last = lax.rem(my + 2, N_DEV)
        out_ref[pl.ds(c_last * M_CHUNK, M_CHUNK), :] = jnp.dot(
            comm_ref[1], w2_ref[...], preferred_element_type=jnp.float32)

    return pl.pallas_call(
        body,
        out_shape=jax.ShapeDtypeStruct((M, F), jnp.float32),
        in_specs=[
            pl.BlockSpec(memory_space=pltpu.ANY),
            pl.BlockSpec(memory_space=pltpu.VMEM),
            pl.BlockSpec(memory_space=pltpu.VMEM),
        ],
        out_specs=pl.BlockSpec(memory_space=pltpu.VMEM),
        scratch_shapes=[
            pltpu.VMEM((M_CHUNK, K), jnp.float32),
            pltpu.VMEM((M_CHUNK, D), jnp.float32),
            pltpu.VMEM((2, M_CHUNK, D), jnp.float32),
            pltpu.VMEM((2, M_CHUNK, D), jnp.float32),
            pltpu.SemaphoreType.DMA,
            pltpu.SemaphoreType.DMA((2,)),
            pltpu.SemaphoreType.DMA((2,)),
            pltpu.SemaphoreType.REGULAR,
        ],
        compiler_params=pltpu.CompilerParams(collective_id=0),
    )(x, W1, W2)


# baseline (device time: 696815 ns/iter reference)
import jax
import jax.numpy as jnp
from jax import lax
from jax.experimental import pallas as pl
from jax.experimental.pallas import tpu as pltpu

N_DEV = 8
M_CHUNK = 1024


def kernel(x, W1, W2):
    M, K = x.shape
    _, D = W1.shape
    _, F = W2.shape
    n_steps = N_DEV - 1

    def body(x_hbm, w1_ref, w2_ref, out_ref,
             xblk_ref, part_ref, sbuf_ref, comm_ref,
             load_sem, send_sems, recv_sems, credit_sem):
        my = lax.axis_index("i")
        left = lax.rem(my + N_DEV - 1, N_DEV)
        right = lax.rem(my + 1, N_DEV)

        barrier = pltpu.get_barrier_semaphore()
        for nbr in (left, right):
            pl.semaphore_signal(barrier, inc=1, device_id=(nbr,),
                                device_id_type=pl.DeviceIdType.MESH)
        pl.semaphore_wait(barrier, 2)

        def compute_partial(c, dst_ref):
            cp = pltpu.make_async_copy(
                x_hbm.at[pl.ds(c * M_CHUNK, M_CHUNK), :], xblk_ref, load_sem)
            cp.start()
            cp.wait()
            dst_ref[...] = jnp.dot(xblk_ref[...], w1_ref[...],
                                   preferred_element_type=jnp.float32)

        def signal_credit():
            pl.semaphore_signal(credit_sem, inc=1, device_id=(left,),
                                device_id_type=pl.DeviceIdType.MESH)

        compute_partial(my, sbuf_ref.at[0])
        for s in range(n_steps):
            slot = s % 2
            if s >= 2:
                pl.semaphore_wait(credit_sem, 1)
            rdma = pltpu.make_async_remote_copy(
                src_ref=sbuf_ref.at[slot],
                dst_ref=comm_ref.at[slot],
                send_sem=send_sems.at[slot],
                recv_sem=recv_sems.at[slot],
                device_id=(right,),
                device_id_type=pl.DeviceIdType.MESH,
            )
            rdma.start()
            c = lax.rem(my + N_DEV - s - 1, N_DEV)
            compute_partial(c, part_ref)
            rdma.wait()
            sbuf_ref[1 - slot] = comm_ref[slot] + part_ref[...]
            signal_credit()

        for t in range(n_steps):
            s = n_steps + t
            slot = s % 2
            pl.semaphore_wait(credit_sem, 1)
            src_ref = sbuf_ref.at[1] if t == 0 else comm_ref.at[(s - 1) % 2]
            rdma = pltpu.make_async_remote_copy(
                src_ref=src_ref,
                dst_ref=comm_ref.at[slot],
                send_sem=send_sems.at[slot],
                recv_sem=recv_sems.at[slot],
                device_id=(right,),
                device_id_type=pl.DeviceIdType.MESH,
            )
            rdma.start()
            c_held = lax.rem(my + 1 - t + N_DEV, N_DEV)
            out_ref[pl.ds(c_held * M_CHUNK, M_CHUNK), :] = jnp.dot(
                src_ref[...], w2_ref[...], preferred_element_type=jnp.float32)
            rdma.wait()
            if 1 <= t <= 5:
                signal_credit()

        c_last = lax.rem(my + 2, N_DEV)
        out_ref[pl.ds(c_last * M_CHUNK, M_CHUNK), :] = jnp.dot(
            comm_ref[1], w2_ref[...], preferred_element_type=jnp.float32)

    return pl.pallas_call(
        body,
        out_shape=jax.ShapeDtypeStruct((M, F), jnp.float32),
        in_specs=[
            pl.BlockSpec(memory_space=pl.ANY),
            pl.BlockSpec(memory_space=pltpu.VMEM),
            pl.BlockSpec(memory_space=pltpu.VMEM),
        ],
        out_specs=pl.BlockSpec(memory_space=pltpu.VMEM),
        scratch_shapes=[
            pltpu.VMEM((M_CHUNK, K), jnp.float32),
            pltpu.VMEM((M_CHUNK, D), jnp.float32),
            pltpu.VMEM((2, M_CHUNK, D), jnp.float32),
            pltpu.VMEM((2, M_CHUNK, D), jnp.float32),
            pltpu.SemaphoreType.DMA,
            pltpu.SemaphoreType.DMA((2,)),
            pltpu.SemaphoreType.DMA((2,)),
            pltpu.SemaphoreType.REGULAR,
        ],
        compiler_params=pltpu.CompilerParams(
            collective_id=0, vmem_limit_bytes=100 * 1024 * 1024),
    )(x, W1, W2)


# device time: 385900 ns/iter; 1.8057x vs baseline; 1.8057x over previous
import jax
import jax.numpy as jnp
from jax import lax
from jax.experimental import pallas as pl
from jax.experimental.pallas import tpu as pltpu

N_DEV = 8
M_CHUNK = 1024
H = M_CHUNK // 2


def kernel(x, W1, W2):
    M, K = x.shape
    _, D = W1.shape
    _, F = W2.shape
    n_steps = N_DEV - 1

    def body(x_hbm, w1_ref, w2_ref, out_ref,
             xblkR, xblkL, partR, partL, sbufR, sbufL, commR, commL,
             loadR, loadL, sendR, recvR, sendL, recvL, credR, credL):
        my = lax.axis_index("i")
        left = lax.rem(my + N_DEV - 1, N_DEV)
        right = lax.rem(my + 1, N_DEV)

        barrier = pltpu.get_barrier_semaphore()
        for nbr in (left, right):
            pl.semaphore_signal(barrier, inc=1, device_id=(nbr,),
                                device_id_type=pl.DeviceIdType.MESH)
        pl.semaphore_wait(barrier, 2)

        def load_and_gemm1(row0, xblk, load_sem, dst_ref):
            cp = pltpu.make_async_copy(
                x_hbm.at[pl.ds(row0, H), :], xblk, load_sem)
            cp.start()
            cp.wait()
            dst_ref[...] = jnp.dot(xblk[...], w1_ref[...],
                                   preferred_element_type=jnp.float32)

        def remote(src, dst, send_sem, recv_sem, dev):
            return pltpu.make_async_remote_copy(
                src_ref=src, dst_ref=dst, send_sem=send_sem,
                recv_sem=recv_sem, device_id=(dev,),
                device_id_type=pl.DeviceIdType.MESH)

        def signal(sem, dev):
            pl.semaphore_signal(sem, inc=1, device_id=(dev,),
                                device_id_type=pl.DeviceIdType.MESH)

        load_and_gemm1(my * M_CHUNK, xblkR, loadR, sbufR.at[0])
        load_and_gemm1(my * M_CHUNK + H, xblkL, loadL, sbufL.at[0])
        for s in range(n_steps):
            slot = s % 2
            if s >= 2:
                pl.semaphore_wait(credR, 1)
                pl.semaphore_wait(credL, 1)
            rR = remote(sbufR.at[slot], commR.at[slot],
                        sendR.at[slot], recvR.at[slot], right)
            rL = remote(sbufL.at[slot], commL.at[slot],
                        sendL.at[slot], recvL.at[slot], left)
            rR.start()
            rL.start()
            cR = lax.rem(my + N_DEV - s - 1, N_DEV)
            cL = lax.rem(my + s + 1, N_DEV)
            load_and_gemm1(cR * M_CHUNK, xblkR, loadR, partR)
            load_and_gemm1(cL * M_CHUNK + H, xblkL, loadL, partL)
            rR.wait()
            rL.wait()
            sbufR[1 - slot] = commR[slot] + partR[...]
            sbufL[1 - slot] = commL[slot] + partL[...]
            signal(credR, left)
            signal(credL, right)

        for t in range(n_steps):
            s = n_steps + t
            slot = s % 2
            pl.semaphore_wait(credR, 1)
            pl.semaphore_wait(credL, 1)
            srcR = sbufR.at[1] if t == 0 else commR.at[(s - 1) % 2]
            srcL = sbufL.at[1] if t == 0 else commL.at[(s - 1) % 2]
            rR = remote(srcR, commR.at[slot], sendR.at[slot],
                        recvR.at[slot], right)
            rL = remote(srcL, commL.at[slot], sendL.at[slot],
                        recvL.at[slot], left)
            rR.start()
            rL.start()
            cRh = lax.rem(my + 1 - t + N_DEV, N_DEV)
            cLh = lax.rem(my - 1 + t + N_DEV, N_DEV)
            out_ref[pl.ds(cRh * M_CHUNK, H), :] = jnp.dot(
                srcR[...], w2_ref[...], preferred_element_type=jnp.float32)
            out_ref[pl.ds(cLh * M_CHUNK + H, H), :] = jnp.dot(
                srcL[...], w2_ref[...], preferred_element_type=jnp.float32)
            rR.wait()
            rL.wait()
            if 1 <= t <= 5:
                signal(credR, left)
                signal(credL, right)

        cRl = lax.rem(my + 2, N_DEV)
        cLl = lax.rem(my + N_DEV - 2, N_DEV)
        out_ref[pl.ds(cRl * M_CHUNK, H), :] = jnp.dot(
            commR[1], w2_ref[...], preferred_element_type=jnp.float32)
        out_ref[pl.ds(cLl * M_CHUNK + H, H), :] = jnp.dot(
            commL[1], w2_ref[...], preferred_element_type=jnp.float32)

    return pl.pallas_call(
        body,
        out_shape=jax.ShapeDtypeStruct((M, F), jnp.float32),
        in_specs=[
            pl.BlockSpec(memory_space=pl.ANY),
            pl.BlockSpec(memory_space=pltpu.VMEM),
            pl.BlockSpec(memory_space=pltpu.VMEM),
        ],
        out_specs=pl.BlockSpec(memory_space=pltpu.VMEM),
        scratch_shapes=[
            pltpu.VMEM((H, K), jnp.float32),
            pltpu.VMEM((H, K), jnp.float32),
            pltpu.VMEM((H, D), jnp.float32),
            pltpu.VMEM((H, D), jnp.float32),
            pltpu.VMEM((2, H, D), jnp.float32),
            pltpu.VMEM((2, H, D), jnp.float32),
            pltpu.VMEM((2, H, D), jnp.float32),
            pltpu.VMEM((2, H, D), jnp.float32),
            pltpu.SemaphoreType.DMA,
            pltpu.SemaphoreType.DMA,
            pltpu.SemaphoreType.DMA((2,)),
            pltpu.SemaphoreType.DMA((2,)),
            pltpu.SemaphoreType.DMA((2,)),
            pltpu.SemaphoreType.DMA((2,)),
            pltpu.SemaphoreType.REGULAR,
            pltpu.SemaphoreType.REGULAR,
        ],
        compiler_params=pltpu.CompilerParams(
            collective_id=0, vmem_limit_bytes=100 * 1024 * 1024),
    )(x, W1, W2)


# device time: 348582 ns/iter; 1.9990x vs baseline; 1.1071x over previous
import jax
import jax.numpy as jnp
from jax import lax
from jax.experimental import pallas as pl
from jax.experimental.pallas import tpu as pltpu

N_DEV = 8
M_CHUNK = 1024
Q = M_CHUNK // 4
N_FLOWS = 4
COMM_DTYPE = jnp.float32


def kernel(x, W1, W2):
    M, K = x.shape
    _, D = W1.shape
    _, F = W2.shape
    n_steps = N_DEV - 1

    def body(x_hbm, w1_ref, w2_ref, out_ref,
             xblk, part, sbuf, comm, *sems):
        load_sems = sems[0:N_FLOWS]
        send_sems = sems[N_FLOWS:2 * N_FLOWS]
        recv_sems = sems[2 * N_FLOWS:3 * N_FLOWS]
        cred_sems = sems[3 * N_FLOWS:4 * N_FLOWS]
        my = lax.axis_index("i")
        left = lax.rem(my + N_DEV - 1, N_DEV)
        right = lax.rem(my + 1, N_DEV)

        barrier = pltpu.get_barrier_semaphore()
        for nbr in (left, right):
            pl.semaphore_signal(barrier, inc=1, device_id=(nbr,),
                                device_id_type=pl.DeviceIdType.MESH)
        pl.semaphore_wait(barrier, 2)

        flow_defs = [(True, 0), (False, 512), (True, 256), (False, 768)]

        class Flow:
            pass

        flows = []
        for fi, (is_r, qoff) in enumerate(flow_defs):
            f = Flow()
            f.fi = fi
            f.qoff = qoff
            f.dst = right if is_r else left
            f.src = left if is_r else right
            if is_r:
                f.rs_chunk = lambda s, my=my: lax.rem(my + N_DEV - s - 1, N_DEV)
                f.ag_chunk = lambda t, my=my: lax.rem(my + N_DEV - t, N_DEV)
                f.own = lax.rem(my + 1, N_DEV)
            else:
                f.rs_chunk = lambda s, my=my: lax.rem(my + s + 1, N_DEV)
                f.ag_chunk = lambda t, my=my: lax.rem(my + t, N_DEV)
                f.own = lax.rem(my + N_DEV - 1, N_DEV)
            f.rdma = {}
            f.load_started = set()
            f.load_idx = 0
            flows.append(f)

        def start_load(f, li, c):
            cp = pltpu.make_async_copy(
                x_hbm.at[pl.ds(c * M_CHUNK + f.qoff, Q), :],
                xblk.at[f.fi, li % 2], load_sems[f.fi].at[li % 2])
            cp.start()
            f.load_started.add(li)
            f.load_cp = getattr(f, 'load_cp', {})
            f.load_cp[li] = cp

        def gemm1(f, li, next_c):
            f.load_cp[li].wait()
            if next_c is not None:
                start_load(f, li + 1, next_c)
            part[f.fi] = jnp.dot(xblk[f.fi, li % 2], w1_ref[...],
                                 preferred_element_type=jnp.float32)

        def S(f, s):
            if s <= n_steps:
                src = sbuf.at[f.fi, s % 2]
            else:
                src = comm.at[f.fi, (s - 1) % 2]
            r = pltpu.make_async_remote_copy(
                src_ref=src, dst_ref=comm.at[f.fi, s % 2],
                send_sem=send_sems[f.fi].at[s % 2],
                recv_sem=recv_sems[f.fi].at[s % 2],
                device_id=(f.dst,), device_id_type=pl.DeviceIdType.MESH)
            r.start()
            f.rdma[s] = r

        def R(f, s):
            f.rdma[s].wait_recv()

        def WS(f, s):
            f.rdma[s].wait_send()

        def C(f):
            pl.semaphore_signal(cred_sems[f.fi], inc=1, device_id=(f.src,),
                                device_id_type=pl.DeviceIdType.MESH)

        def K(f):
            pl.semaphore_wait(cred_sems[f.fi], 1)

        def G(f, c, src):
            out_ref[pl.ds(c * M_CHUNK + f.qoff, Q), :] = jnp.dot(
                src[...].astype(jnp.float32), w2_ref[...],
                preferred_element_type=jnp.float32)

        for f in flows:
            start_load(f, 0, my)
        for f in flows:
            gemm1(f, 0, f.rs_chunk(0))
            sbuf[f.fi, 0] = part[f.fi].astype(COMM_DTYPE)
            S(f, 0)

        for s in range(n_steps):
            for f in flows:
                nxt = f.rs_chunk(s + 1) if s + 1 < n_steps else None
                gemm1(f, s + 1, nxt)
                R(f, s)
                if s >= 1:
                    WS(f, s - 1)
                sbuf[f.fi, (s + 1) % 2] = (
                    comm[f.fi, s % 2].astype(jnp.float32) + part[f.fi]
                ).astype(COMM_DTYPE)
                C(f)
                if s < n_steps - 1:
                    if s + 1 >= 2:
                        K(f)
                    S(f, s + 1)

        for f in flows:
            K(f)
            S(f, n_steps)
            WS(f, n_steps - 1)
            G(f, f.own, sbuf.at[f.fi, 1])

        for t in range(n_steps):
            for f in flows:
                s = n_steps + t
                R(f, s)
                WS(f, s)
                if 1 <= t <= n_steps - 2:
                    C(f)
                if t < n_steps - 1:
                    K(f)
                    S(f, s + 1)
                G(f, f.ag_chunk(t), comm.at[f.fi, s % 2])

    return pl.pallas_call(
        body,
        out_shape=jax.ShapeDtypeStruct((M, F), jnp.float32),
        in_specs=[
            pl.BlockSpec(memory_space=pl.ANY),
            pl.BlockSpec(memory_space=pltpu.VMEM),
            pl.BlockSpec(memory_space=pltpu.VMEM),
        ],
        out_specs=pl.BlockSpec(memory_space=pltpu.VMEM),
        scratch_shapes=[
            pltpu.VMEM((N_FLOWS, 2, Q, K), jnp.float32),
            pltpu.VMEM((N_FLOWS, Q, D), jnp.float32),
            pltpu.VMEM((N_FLOWS, 2, Q, D), COMM_DTYPE),
            pltpu.VMEM((N_FLOWS, 2, Q, D), COMM_DTYPE),
            *([pltpu.SemaphoreType.DMA((2,))] * N_FLOWS),
            *([pltpu.SemaphoreType.DMA((2,))] * N_FLOWS),
            *([pltpu.SemaphoreType.DMA((2,))] * N_FLOWS),
            *([pltpu.SemaphoreType.REGULAR] * N_FLOWS),
        ],
        compiler_params=pltpu.CompilerParams(
            collective_id=0, vmem_limit_bytes=100 * 1024 * 1024),
    )(x, W1, W2)


# device time: 191419 ns/iter; 3.6403x vs baseline; 1.8210x over previous
import jax
import jax.numpy as jnp
from jax import lax
from jax.experimental import pallas as pl
from jax.experimental.pallas import tpu as pltpu

N_DEV = 8
M_CHUNK = 1024
Q = M_CHUNK // 4
N_FLOWS = 4
COMM_DTYPE = jnp.bfloat16


def kernel(x, W1, W2):
    M, K = x.shape
    _, D = W1.shape
    _, F = W2.shape
    n_steps = N_DEV - 1

    def body(x_hbm, w1_ref, w2_ref, out_ref,
             xblk, part, sbuf, comm, *sems):
        load_sems = sems[0:N_FLOWS]
        send_sems = sems[N_FLOWS:2 * N_FLOWS]
        recv_sems = sems[2 * N_FLOWS:3 * N_FLOWS]
        cred_sems = sems[3 * N_FLOWS:4 * N_FLOWS]
        my = lax.axis_index("i")
        left = lax.rem(my + N_DEV - 1, N_DEV)
        right = lax.rem(my + 1, N_DEV)

        barrier = pltpu.get_barrier_semaphore()
        for nbr in (left, right):
            pl.semaphore_signal(barrier, inc=1, device_id=(nbr,),
                                device_id_type=pl.DeviceIdType.MESH)
        pl.semaphore_wait(barrier, 2)

        flow_defs = [(True, 0), (False, 512), (True, 256), (False, 768)]

        class Flow:
            pass

        flows = []
        for fi, (is_r, qoff) in enumerate(flow_defs):
            f = Flow()
            f.fi = fi
            f.qoff = qoff
            f.dst = right if is_r else left
            f.src = left if is_r else right
            if is_r:
                f.rs_chunk = lambda s, my=my: lax.rem(my + N_DEV - s - 1, N_DEV)
                f.ag_chunk = lambda t, my=my: lax.rem(my + N_DEV - t, N_DEV)
                f.own = lax.rem(my + 1, N_DEV)
            else:
                f.rs_chunk = lambda s, my=my: lax.rem(my + s + 1, N_DEV)
                f.ag_chunk = lambda t, my=my: lax.rem(my + t, N_DEV)
                f.own = lax.rem(my + N_DEV - 1, N_DEV)
            f.rdma = {}
            f.load_started = set()
            f.load_idx = 0
            flows.append(f)

        def start_load(f, li, c):
            cp = pltpu.make_async_copy(
                x_hbm.at[pl.ds(c * M_CHUNK + f.qoff, Q), :],
                xblk.at[f.fi, li % 2], load_sems[f.fi].at[li % 2])
            cp.start()
            f.load_started.add(li)
            f.load_cp = getattr(f, 'load_cp', {})
            f.load_cp[li] = cp

        def gemm1(f, li, next_c):
            f.load_cp[li].wait()
            if next_c is not None:
                start_load(f, li + 1, next_c)
            part[f.fi] = jnp.dot(xblk[f.fi, li % 2], w1_ref[...],
                                 preferred_element_type=jnp.float32)

        def S(f, s):
            if s <= n_steps:
                src = sbuf.at[f.fi, s % 2]
            else:
                src = comm.at[f.fi, (s - 1) % 2]
            r = pltpu.make_async_remote_copy(
                src_ref=src, dst_ref=comm.at[f.fi, s % 2],
                send_sem=send_sems[f.fi].at[s % 2],
                recv_sem=recv_sems[f.fi].at[s % 2],
                device_id=(f.dst,), device_id_type=pl.DeviceIdType.MESH)
            r.start()
            f.rdma[s] = r

        def R(f, s):
            f.rdma[s].wait_recv()

        def WS(f, s):
            f.rdma[s].wait_send()

        def C(f):
            pl.semaphore_signal(cred_sems[f.fi], inc=1, device_id=(f.src,),
                                device_id_type=pl.DeviceIdType.MESH)

        def K(f):
            pl.semaphore_wait(cred_sems[f.fi], 1)

        def G(f, c, src):
            out_ref[pl.ds(c * M_CHUNK + f.qoff, Q), :] = jnp.dot(
                src[...].astype(jnp.float32), w2_ref[...],
                preferred_element_type=jnp.float32)

        for f in flows:
            start_load(f, 0, my)
        for f in flows:
            gemm1(f, 0, f.rs_chunk(0))
            sbuf[f.fi, 0] = part[f.fi].astype(COMM_DTYPE)
            S(f, 0)

        for s in range(n_steps):
            for f in flows:
                nxt = f.rs_chunk(s + 1) if s + 1 < n_steps else None
                gemm1(f, s + 1, nxt)
                R(f, s)
                if s >= 1:
                    WS(f, s - 1)
                sbuf[f.fi, (s + 1) % 2] = (
                    comm[f.fi, s % 2].astype(jnp.float32) + part[f.fi]
                ).astype(COMM_DTYPE)
                C(f)
                if s < n_steps - 1:
                    if s + 1 >= 2:
                        K(f)
                    S(f, s + 1)

        for f in flows:
            K(f)
            S(f, n_steps)
            WS(f, n_steps - 1)
            G(f, f.own, sbuf.at[f.fi, 1])

        for t in range(n_steps):
            for f in flows:
                s = n_steps + t
                R(f, s)
                WS(f, s)
                if 1 <= t <= n_steps - 2:
                    C(f)
                if t < n_steps - 1:
                    K(f)
                    S(f, s + 1)
                G(f, f.ag_chunk(t), comm.at[f.fi, s % 2])

    return pl.pallas_call(
        body,
        out_shape=jax.ShapeDtypeStruct((M, F), jnp.float32),
        in_specs=[
            pl.BlockSpec(memory_space=pl.ANY),
            pl.BlockSpec(memory_space=pltpu.VMEM),
            pl.BlockSpec(memory_space=pltpu.VMEM),
        ],
        out_specs=pl.BlockSpec(memory_space=pltpu.VMEM),
        scratch_shapes=[
            pltpu.VMEM((N_FLOWS, 2, Q, K), jnp.float32),
            pltpu.VMEM((N_FLOWS, Q, D), jnp.float32),
            pltpu.VMEM((N_FLOWS, 2, Q, D), COMM_DTYPE),
            pltpu.VMEM((N_FLOWS, 2, Q, D), COMM_DTYPE),
            *([pltpu.SemaphoreType.DMA((2,))] * N_FLOWS),
            *([pltpu.SemaphoreType.DMA((2,))] * N_FLOWS),
            *([pltpu.SemaphoreType.DMA((2,))] * N_FLOWS),
            *([pltpu.SemaphoreType.REGULAR] * N_FLOWS),
        ],
        compiler_params=pltpu.CompilerParams(
            collective_id=0, vmem_limit_bytes=100 * 1024 * 1024),
    )(x, W1, W2)


# device time: 190399 ns/iter; 3.6598x vs baseline; 1.0054x over previous
import jax
import jax.numpy as jnp
from jax import lax
from jax.experimental import pallas as pl
from jax.experimental.pallas import tpu as pltpu

N_DEV = 8
M_CHUNK = 1024
Q = M_CHUNK // 4
N_FLOWS = 4
COMM_DTYPE = jnp.bfloat16


def kernel(x, W1, W2):
    M, K = x.shape
    _, D = W1.shape
    _, F = W2.shape
    n_steps = N_DEV - 1

    def body(x_hbm, w1_ref, w2_ref, out_ref,
             xblk, part, sbuf, comm, w1b, w2b, *sems):
        load_sems = sems[0:N_FLOWS]
        send_sems = sems[N_FLOWS:2 * N_FLOWS]
        recv_sems = sems[2 * N_FLOWS:3 * N_FLOWS]
        cred_sems = sems[3 * N_FLOWS:4 * N_FLOWS]
        my = lax.axis_index("i")
        left = lax.rem(my + N_DEV - 1, N_DEV)
        right = lax.rem(my + 1, N_DEV)

        w1b[...] = w1_ref[...].astype(jnp.bfloat16)
        w2b[...] = w2_ref[...].astype(jnp.bfloat16)

        barrier = pltpu.get_barrier_semaphore()
        for nbr in (left, right):
            pl.semaphore_signal(barrier, inc=1, device_id=(nbr,),
                                device_id_type=pl.DeviceIdType.MESH)
        pl.semaphore_wait(barrier, 2)

        flow_defs = [(True, 0), (False, 512), (True, 256), (False, 768)]

        class Flow:
            pass

        flows = []
        for fi, (is_r, qoff) in enumerate(flow_defs):
            f = Flow()
            f.fi = fi
            f.qoff = qoff
            f.dst = right if is_r else left
            f.src = left if is_r else right
            if is_r:
                f.rs_chunk = lambda s, my=my: lax.rem(my + N_DEV - s - 1, N_DEV)
                f.ag_chunk = lambda t, my=my: lax.rem(my + N_DEV - t, N_DEV)
                f.own = lax.rem(my + 1, N_DEV)
            else:
                f.rs_chunk = lambda s, my=my: lax.rem(my + s + 1, N_DEV)
                f.ag_chunk = lambda t, my=my: lax.rem(my + t, N_DEV)
                f.own = lax.rem(my + N_DEV - 1, N_DEV)
            f.rdma = {}
            f.load_started = set()
            f.load_idx = 0
            flows.append(f)

        def start_load(f, li, c):
            cp = pltpu.make_async_copy(
                x_hbm.at[pl.ds(c * M_CHUNK + f.qoff, Q), :],
                xblk.at[f.fi, li % 2], load_sems[f.fi].at[li % 2])
            cp.start()
            f.load_started.add(li)
            f.load_cp = getattr(f, 'load_cp', {})
            f.load_cp[li] = cp

        def gemm1(f, li, next_c):
            f.load_cp[li].wait()
            if next_c is not None:
                start_load(f, li + 1, next_c)
            part[f.fi] = jnp.dot(xblk[f.fi, li % 2].astype(jnp.bfloat16),
                                 w1b[...], preferred_element_type=jnp.float32)

        def S(f, s):
            if s <= n_steps:
                src = sbuf.at[f.fi, s % 2]
            else:
                src = comm.at[f.fi, (s - 1) % 2]
            r = pltpu.make_async_remote_copy(
                src_ref=src, dst_ref=comm.at[f.fi, s % 2],
                send_sem=send_sems[f.fi].at[s % 2],
                recv_sem=recv_sems[f.fi].at[s % 2],
                device_id=(f.dst,), device_id_type=pl.DeviceIdType.MESH)
            r.start()
            f.rdma[s] = r

        def R(f, s):
            f.rdma[s].wait_recv()

        def WS(f, s):
            f.rdma[s].wait_send()

        def C(f):
            pl.semaphore_signal(cred_sems[f.fi], inc=1, device_id=(f.src,),
                                device_id_type=pl.DeviceIdType.MESH)

        def K(f):
            pl.semaphore_wait(cred_sems[f.fi], 1)

        def G(f, c, src):
            out_ref[pl.ds(c * M_CHUNK + f.qoff, Q), :] = jnp.dot(
                src[...].astype(jnp.bfloat16), w2b[...],
                preferred_element_type=jnp.float32)

        for f in flows:
            start_load(f, 0, my)
        for f in flows:
            gemm1(f, 0, f.rs_chunk(0))
            sbuf[f.fi, 0] = part[f.fi].astype(COMM_DTYPE)
            S(f, 0)

        for s in range(n_steps):
            for f in flows:
                nxt = f.rs_chunk(s + 1) if s + 1 < n_steps else None
                gemm1(f, s + 1, nxt)
                R(f, s)
                if s >= 1:
                    WS(f, s - 1)
                sbuf[f.fi, (s + 1) % 2] = (
                    comm[f.fi, s % 2].astype(jnp.float32) + part[f.fi]
                ).astype(COMM_DTYPE)
                C(f)
                if s < n_steps - 1:
                    if s + 1 >= 2:
                        K(f)
                    S(f, s + 1)

        for f in flows:
            K(f)
            S(f, n_steps)
            WS(f, n_steps - 1)
            G(f, f.own, sbuf.at[f.fi, 1])

        for t in range(n_steps):
            for f in flows:
                s = n_steps + t
                R(f, s)
                WS(f, s)
                if 1 <= t <= n_steps - 2:
                    C(f)
                if t < n_steps - 1:
                    K(f)
                    S(f, s + 1)
                G(f, f.ag_chunk(t), comm.at[f.fi, s % 2])

    return pl.pallas_call(
        body,
        out_shape=jax.ShapeDtypeStruct((M, F), jnp.float32),
        in_specs=[
            pl.BlockSpec(memory_space=pl.ANY),
            pl.BlockSpec(memory_space=pltpu.VMEM),
            pl.BlockSpec(memory_space=pltpu.VMEM),
        ],
        out_specs=pl.BlockSpec(memory_space=pltpu.VMEM),
        scratch_shapes=[
            pltpu.VMEM((N_FLOWS, 2, Q, K), jnp.float32),
            pltpu.VMEM((N_FLOWS, Q, D), jnp.float32),
            pltpu.VMEM((N_FLOWS, 2, Q, D), COMM_DTYPE),
            pltpu.VMEM((N_FLOWS, 2, Q, D), COMM_DTYPE),
            pltpu.VMEM((K, D), jnp.bfloat16),
            pltpu.VMEM((D, F), jnp.bfloat16),
            *([pltpu.SemaphoreType.DMA((2,))] * N_FLOWS),
            *([pltpu.SemaphoreType.DMA((2,))] * N_FLOWS),
            *([pltpu.SemaphoreType.DMA((2,))] * N_FLOWS),
            *([pltpu.SemaphoreType.REGULAR] * N_FLOWS),
        ],
        compiler_params=pltpu.CompilerParams(
            collective_id=0, vmem_limit_bytes=100 * 1024 * 1024),
    )(x, W1, W2)
